# baseline (speedup 1.0000x reference)
import functools

import jax
import jax.numpy as jnp
from jax import lax
from jax.experimental import pallas as pl
from jax.experimental.pallas import tpu as pltpu
from jax.experimental.pallas import tpu_sc as plsc

N_NODES = 100000
TYPE_NUM = 300
DIM = 128

NC = 2
NS = 16
LANES = 16
NW = NC * NS

CHUNK = 128
N_CHUNKS = 25
PER_W = CHUNK * N_CHUNKS
N_PAD = PER_W * NW
NBUF = 4
GROUPS = CHUNK // LANES


def _body(nt_hbm, ls_hbm, w_hbm, out_hbm, nt_v, ls_v, w_v, rows_v,
          sem_w):
    wid = lax.axis_index("s") * NC + lax.axis_index("c")
    base = wid * PER_W

    pltpu.sync_copy(w_hbm, w_v)
    pltpu.sync_copy(nt_hbm.at[pl.ds(base, PER_W)], nt_v)
    pltpu.sync_copy(ls_hbm.at[pl.ds(base, PER_W)], ls_v)

    lane = lax.iota(jnp.int32, 16)

    def write_cp(j, b):
        return pltpu.make_async_copy(
            rows_v.at[pl.ds(b * CHUNK * DIM, CHUNK * DIM)],
            out_hbm.at[pl.ds((base + j * CHUNK) * DIM, CHUNK * DIM)],
            sem_w)

    def do_chunk(j, _):
        b = lax.rem(j, NBUF)

        @pl.when(j >= NBUF)
        def _():
            write_cp(j, b).wait()

        def do_group(g, _):
            off = j * CHUNK + g * LANES
            nt = nt_v[pl.ds(off, LANES)]
            l = ls_v[pl.ds(off, LANES)]
            gbase = lax.shift_left(nt + l * 100, 7)
            dbase = lax.shift_left((b * CHUNK) + g * LANES + lane, 7)
            for c in range(DIM):
                vals = plsc.load_gather(w_v, [gbase + c])
                plsc.store_scatter(rows_v, [dbase + c], vals)
            return 0

        lax.fori_loop(0, GROUPS, do_group, 0)
        write_cp(j, b).start()
        return 0

    lax.fori_loop(0, N_CHUNKS, do_chunk, 0)

    for k in range(NBUF):
        write_cp(N_CHUNKS - NBUF + k, k).wait()


_sc_embed = functools.partial(
    pl.kernel,
    mesh=plsc.VectorSubcoreMesh(core_axis_name="c", subcore_axis_name="s"),
    out_type=jax.ShapeDtypeStruct((N_PAD * DIM,), jnp.float32),
    scratch_types=[
        pltpu.VMEM((PER_W,), jnp.int32),
        pltpu.VMEM((PER_W,), jnp.int32),
        pltpu.VMEM((TYPE_NUM * DIM,), jnp.float32),
        pltpu.VMEM((NBUF * CHUNK * DIM,), jnp.float32),
        pltpu.SemaphoreType.DMA,
    ],
    compiler_params=pltpu.CompilerParams(needs_layout_passes=False),
)(_body)


def kernel(node_type, ls, weight):
    pad = N_PAD - N_NODES
    nt = jnp.pad(node_type, (0, pad))
    lsp = jnp.pad(ls, (0, pad))
    out = _sc_embed(nt, lsp, weight.reshape(-1))
    return out[:N_NODES * DIM].reshape(N_NODES, DIM)


# speedup vs baseline: 3.4720x; 3.4720x over previous
import functools

import jax
import jax.numpy as jnp
from jax import lax
from jax.experimental import pallas as pl
from jax.experimental.pallas import tpu as pltpu
from jax.experimental.pallas import tpu_sc as plsc

N_NODES = 100000
TYPE_NUM = 300
DIM = 128

NC = 2
NS = 16
LANES = 16
NW = NC * NS

CHUNK = 128
N_CHUNKS = 25
PER_W = CHUNK * N_CHUNKS
N_PAD = PER_W * NW
NBUF = 4
GROUPS = CHUNK // LANES


def _body(nt_hbm, ls_hbm, w_hbm, out_hbm, nt_v, ls_v, w_v, rows_v,
          sem_w):
    wid = lax.axis_index("s") * NC + lax.axis_index("c")
    base = wid * PER_W

    pltpu.sync_copy(w_hbm, w_v)
    pltpu.sync_copy(nt_hbm.at[pl.ds(base, PER_W)], nt_v)
    pltpu.sync_copy(ls_hbm.at[pl.ds(base, PER_W)], ls_v)

    def write_cp(j, b):
        return pltpu.make_async_copy(
            rows_v.at[pl.ds(b * CHUNK * DIM, CHUNK * DIM)],
            out_hbm.at[pl.ds((base + j * CHUNK) * DIM, CHUNK * DIM)],
            sem_w)

    def do_chunk(j, _):
        b = lax.rem(j, NBUF)

        @pl.when(j >= NBUF)
        def _():
            write_cp(j, b).wait()

        def do_group(g, _):
            off = j * CHUNK + g * LANES
            nt = nt_v[pl.ds(off, LANES)]
            l = ls_v[pl.ds(off, LANES)]
            srcs = lax.shift_left(nt + l * 100, 7)
            dst0 = lax.shift_left(b * CHUNK + g * LANES, 7)
            for r in range(LANES):
                src = srcs[r]
                dst = dst0 + r * DIM
                for k in range(0, DIM, LANES):
                    rows_v[pl.ds(dst + k, LANES)] = w_v[pl.ds(src + k, LANES)]
            return 0

        lax.fori_loop(0, GROUPS, do_group, 0)
        write_cp(j, b).start()
        return 0

    lax.fori_loop(0, N_CHUNKS, do_chunk, 0)

    for k in range(NBUF):
        write_cp(N_CHUNKS - NBUF + k, k).wait()


_sc_embed = functools.partial(
    pl.kernel,
    mesh=plsc.VectorSubcoreMesh(core_axis_name="c", subcore_axis_name="s"),
    out_type=jax.ShapeDtypeStruct((N_PAD * DIM,), jnp.float32),
    scratch_types=[
        pltpu.VMEM((PER_W,), jnp.int32),
        pltpu.VMEM((PER_W,), jnp.int32),
        pltpu.VMEM((TYPE_NUM * DIM,), jnp.float32),
        pltpu.VMEM((NBUF * CHUNK * DIM,), jnp.float32),
        pltpu.SemaphoreType.DMA,
    ],
    compiler_params=pltpu.CompilerParams(needs_layout_passes=False),
)(_body)


def kernel(node_type, ls, weight):
    pad = N_PAD - N_NODES
    nt = jnp.pad(node_type, (0, pad))
    lsp = jnp.pad(ls, (0, pad))
    out = _sc_embed(nt, lsp, weight.reshape(-1))
    return out[:N_NODES * DIM].reshape(N_NODES, DIM)


# speedup vs baseline: 3.5407x; 1.0198x over previous
import functools

import jax
import jax.numpy as jnp
from jax import lax
from jax.experimental import pallas as pl
from jax.experimental.pallas import tpu as pltpu
from jax.experimental.pallas import tpu_sc as plsc

N_NODES = 100000
TYPE_NUM = 300
DIM = 128

NC = 2
NS = 16
LANES = 16
NW = NC * NS

COUNT = 3128
CHUNK = 136
N_CHUNKS = 23
NBUF = 4
GROUP_OFFS = tuple(range(0, CHUNK - LANES, LANES)) + (CHUNK - LANES,)


def _body(nt_hbm, ls_hbm, w_hbm, out_hbm, nt_v, ls_v, w_v, rows_v,
          sem_w):
    wid = lax.axis_index("s") * NC + lax.axis_index("c")
    base = lax.min(wid * COUNT, N_NODES - COUNT)

    pltpu.sync_copy(w_hbm, w_v)
    pltpu.sync_copy(nt_hbm.at[pl.ds(base, COUNT)], nt_v)
    pltpu.sync_copy(ls_hbm.at[pl.ds(base, COUNT)], ls_v)

    def write_cp(j, b):
        return pltpu.make_async_copy(
            rows_v.at[pl.ds(b * CHUNK * DIM, CHUNK * DIM)],
            out_hbm.at[pl.ds((base + j * CHUNK) * DIM, CHUNK * DIM)],
            sem_w)

    def do_chunk(j, _):
        b = lax.rem(j, NBUF)

        @pl.when(j >= NBUF)
        def _():
            write_cp(j, b).wait()

        for goff in GROUP_OFFS:
            off = j * CHUNK + goff
            nt = nt_v[pl.ds(off, LANES)]
            l = ls_v[pl.ds(off, LANES)]
            srcs = lax.shift_left(nt + l * 100, 7)
            dst0 = (b * CHUNK + goff) * DIM
            for r in range(LANES):
                src = srcs[r]
                dst = dst0 + r * DIM
                for k in range(0, DIM, LANES):
                    rows_v[pl.ds(dst + k, LANES)] = w_v[pl.ds(src + k, LANES)]

        write_cp(j, b).start()
        return 0

    lax.fori_loop(0, N_CHUNKS, do_chunk, 0)

    for k in range(NBUF):
        write_cp(N_CHUNKS - NBUF + k, k).wait()


_sc_embed = functools.partial(
    pl.kernel,
    mesh=plsc.VectorSubcoreMesh(core_axis_name="c", subcore_axis_name="s"),
    out_type=jax.ShapeDtypeStruct((N_NODES * DIM,), jnp.float32),
    scratch_types=[
        pltpu.VMEM((COUNT,), jnp.int32),
        pltpu.VMEM((COUNT,), jnp.int32),
        pltpu.VMEM((TYPE_NUM * DIM,), jnp.float32),
        pltpu.VMEM((NBUF * CHUNK * DIM,), jnp.float32),
        pltpu.SemaphoreType.DMA,
    ],
    compiler_params=pltpu.CompilerParams(needs_layout_passes=False),
)(_body)


def kernel(node_type, ls, weight):
    out = _sc_embed(node_type, ls, weight.reshape(-1))
    return out.reshape(N_NODES, DIM)


# speedup vs baseline: 3.6771x; 1.0385x over previous
import functools

import jax
import jax.numpy as jnp
from jax import lax
from jax.experimental import pallas as pl
from jax.experimental.pallas import tpu as pltpu
from jax.experimental.pallas import tpu_sc as plsc

N_NODES = 100000
TYPE_NUM = 300
DIM = 128

NC = 2
NS = 16
LANES = 16
NW = NC * NS

COUNT = 3128
CHUNK = 128
N_CHUNKS = 25
LAST_CB = COUNT - CHUNK
NBUF = 4
GROUP_OFFS = tuple(range(0, CHUNK, LANES))


def _body(nt_hbm, ls_hbm, w_hbm, out_hbm, nt_v, ls_v, w_v, rows_v,
          sem_w):
    wid = lax.axis_index("s") * NC + lax.axis_index("c")
    base = lax.min(wid * COUNT, N_NODES - COUNT)

    pltpu.sync_copy(w_hbm, w_v)
    pltpu.sync_copy(nt_hbm.at[pl.ds(base, COUNT)], nt_v)
    pltpu.sync_copy(ls_hbm.at[pl.ds(base, COUNT)], ls_v)

    def write_cp(j, b):
        cb = lax.min(j * CHUNK, LAST_CB)
        return pltpu.make_async_copy(
            rows_v.at[pl.ds(b * CHUNK * DIM, CHUNK * DIM)],
            out_hbm.at[pl.ds((base + cb) * DIM, CHUNK * DIM)],
            sem_w)

    def do_chunk(j, _):
        b = lax.rem(j, NBUF)

        @pl.when(j >= NBUF)
        def _():
            write_cp(j, b).wait()

        cb = lax.min(j * CHUNK, LAST_CB)
        for goff in GROUP_OFFS:
            off = cb + goff
            nt = nt_v[pl.ds(off, LANES)]
            l = ls_v[pl.ds(off, LANES)]
            srcs = lax.shift_left(nt + l * 100, 7)
            dst0 = (b * CHUNK + goff) * DIM
            for r in range(LANES):
                src = srcs[r]
                dst = dst0 + r * DIM
                for k in range(0, DIM, LANES):
                    rows_v[pl.ds(dst + k, LANES)] = w_v[pl.ds(src + k, LANES)]

        write_cp(j, b).start()
        return 0

    lax.fori_loop(0, N_CHUNKS, do_chunk, 0)

    for k in range(NBUF):
        write_cp(N_CHUNKS - NBUF + k, k).wait()


_sc_embed = functools.partial(
    pl.kernel,
    mesh=plsc.VectorSubcoreMesh(core_axis_name="c", subcore_axis_name="s"),
    out_type=jax.ShapeDtypeStruct((N_NODES * DIM,), jnp.float32),
    scratch_types=[
        pltpu.VMEM((COUNT,), jnp.int32),
        pltpu.VMEM((COUNT,), jnp.int32),
        pltpu.VMEM((TYPE_NUM * DIM,), jnp.float32),
        pltpu.VMEM((NBUF * CHUNK * DIM,), jnp.float32),
        pltpu.SemaphoreType.DMA,
    ],
    compiler_params=pltpu.CompilerParams(needs_layout_passes=False),
)(_body)


def kernel(node_type, ls, weight):
    out = _sc_embed(node_type, ls, weight.reshape(-1))
    return out.reshape(N_NODES, DIM)


# speedup vs baseline: 4.6059x; 1.2526x over previous
import functools

import jax
import jax.numpy as jnp
from jax import lax
from jax.experimental import pallas as pl
from jax.experimental.pallas import tpu as pltpu
from jax.experimental.pallas import tpu_sc as plsc

N_NODES = 100000
TYPE_NUM = 300
DIM = 128

NC = 2
NS = 16
LANES = 16
NW = NC * NS

COUNT = 3200
CHUNK = 128
N_CHUNKS = COUNT // CHUNK
NBUF = 4
LAG = 2


def _body(nt_hbm, ls_hbm, w_hbm, out_hbm, idx_v, ls_v, rows_v, *sems):
    sems_g = sems[:NBUF]
    sems_w = sems[NBUF:]

    wid = lax.axis_index("s") * NC + lax.axis_index("c")
    base = lax.min(wid * COUNT, N_NODES - COUNT)

    pltpu.sync_copy(nt_hbm.at[pl.ds(base, COUNT)], idx_v)
    pltpu.sync_copy(ls_hbm.at[pl.ds(base, COUNT)], ls_v)

    def fuse(t, _):
        off = t * LANES
        idx_v[pl.ds(off, LANES)] = (
            idx_v[pl.ds(off, LANES)] + ls_v[pl.ds(off, LANES)] * 100)
        return 0

    lax.fori_loop(0, COUNT // LANES, fuse, 0)

    def gather_cp(j):
        b = j % NBUF
        return pltpu.make_async_copy(
            w_hbm.at[idx_v.at[pl.ds(j * CHUNK, CHUNK)]],
            rows_v.at[b],
            sems_g[b])

    def write_cp(j):
        b = j % NBUF
        return pltpu.make_async_copy(
            rows_v.at[b],
            out_hbm.at[pl.ds(base + j * CHUNK, CHUNK)],
            sems_w[b])

    for j in range(N_CHUNKS + LAG):
        if j < N_CHUNKS:
            if j >= NBUF:
                write_cp(j - NBUF).wait()
            gather_cp(j).start()
        if j >= LAG:
            i = j - LAG
            gather_cp(i).wait()
            write_cp(i).start()

    for i in range(N_CHUNKS - NBUF, N_CHUNKS):
        write_cp(i).wait()


_sc_embed = functools.partial(
    pl.kernel,
    mesh=plsc.VectorSubcoreMesh(core_axis_name="c", subcore_axis_name="s"),
    out_type=jax.ShapeDtypeStruct((N_NODES, DIM), jnp.float32),
    scratch_types=[
        pltpu.VMEM((COUNT,), jnp.int32),
        pltpu.VMEM((COUNT,), jnp.int32),
        pltpu.VMEM((NBUF, CHUNK, DIM), jnp.float32),
    ] + [pltpu.SemaphoreType.DMA] * (2 * NBUF),
    compiler_params=pltpu.CompilerParams(needs_layout_passes=False),
)(_body)


def kernel(node_type, ls, weight):
    return _sc_embed(node_type, ls, weight)


# speedup vs baseline: 12.3176x; 2.6743x over previous
import functools

import jax
import jax.numpy as jnp
from jax import lax
from jax.experimental import pallas as pl
from jax.experimental.pallas import tpu as pltpu
from jax.experimental.pallas import tpu_sc as plsc

N_NODES = 100000
TYPE_NUM = 300
DIM = 128

NC = 2
NS = 16
LANES = 16
NW = NC * NS

COUNT = 3200
CHUNK = 128
N_CHUNKS = COUNT // CHUNK
NBUF = 4
LAG = 2


def _body(nt_hbm, ls_hbm, w_hbm, out_hbm, idx_v, ls_v, w_v, rows_v, *sems):
    sems_g = sems[:NBUF]
    sems_w = sems[NBUF:]

    sid = lax.axis_index("s")
    wid = sid * NC + lax.axis_index("c")
    base = lax.min(wid * COUNT, N_NODES - COUNT)

    @pl.when(sid == 0)
    def _():
        pltpu.sync_copy(w_hbm, w_v)

    pltpu.sync_copy(nt_hbm.at[pl.ds(base, COUNT)], idx_v)
    pltpu.sync_copy(ls_hbm.at[pl.ds(base, COUNT)], ls_v)
    plsc.subcore_barrier()

    def fuse(t, _):
        off = t * LANES
        idx_v[pl.ds(off, LANES)] = (
            idx_v[pl.ds(off, LANES)] + ls_v[pl.ds(off, LANES)] * 100)
        return 0

    lax.fori_loop(0, COUNT // LANES, fuse, 0)

    def gather_cp(j):
        b = j % NBUF
        return pltpu.make_async_copy(
            w_v.at[idx_v.at[pl.ds(j * CHUNK, CHUNK)]],
            rows_v.at[b],
            sems_g[b])

    def write_cp(j):
        b = j % NBUF
        return pltpu.make_async_copy(
            rows_v.at[b],
            out_hbm.at[pl.ds(base + j * CHUNK, CHUNK)],
            sems_w[b])

    for j in range(N_CHUNKS + LAG):
        if j < N_CHUNKS:
            if j >= NBUF:
                write_cp(j - NBUF).wait()
            gather_cp(j).start()
        if j >= LAG:
            i = j - LAG
            gather_cp(i).wait()
            write_cp(i).start()

    for i in range(N_CHUNKS - NBUF, N_CHUNKS):
        write_cp(i).wait()


_sc_embed = functools.partial(
    pl.kernel,
    mesh=plsc.VectorSubcoreMesh(core_axis_name="c", subcore_axis_name="s"),
    out_type=jax.ShapeDtypeStruct((N_NODES, DIM), jnp.float32),
    scratch_types=[
        pltpu.VMEM((COUNT,), jnp.int32),
        pltpu.VMEM((COUNT,), jnp.int32),
        pltpu.VMEM_SHARED((TYPE_NUM, DIM), jnp.float32),
        pltpu.VMEM((NBUF, CHUNK, DIM), jnp.float32),
    ] + [pltpu.SemaphoreType.DMA] * (2 * NBUF),
    compiler_params=pltpu.CompilerParams(needs_layout_passes=False),
)(_body)


def kernel(node_type, ls, weight):
    return _sc_embed(node_type, ls, weight)
